# baseline (speedup 1.0000x reference)
import jax
import jax.numpy as jnp
from jax.experimental import pallas as pl
from jax.experimental.pallas import tpu as pltpu

BEAMS = 8
GROUPS = 32
GPB = 4
ROWS = BEAMS * GPB
VOCAB = 100000
CHUNK = 512
WIN = 544
NCHUNK = (VOCAB + CHUNK - 1) // CHUNK
NEG_INF = float("-inf")
BIG_I32 = 2**31 - 1


def _fused(past_ref, cur_lo_ref, cur_hi_ref, topv_ref, sym_ref):
    half = ROWS // 2
    maxes = []
    for c in range(NCHUNK):
        lo = c * CHUNK
        hi = min(lo + CHUNK, VOCAB)
        maxes.append(jnp.concatenate(
            [jnp.max(cur_lo_ref[:, lo:hi], axis=1, keepdims=True),
             jnp.max(cur_hi_ref[:, lo:hi], axis=1, keepdims=True)], axis=0))
    past = past_ref[...]
    cmax_all = jnp.concatenate(maxes, axis=1) + past

    pid = (jax.lax.broadcasted_iota(jnp.int32, (BEAMS, NCHUNK), 0) * NCHUNK
           + jax.lax.broadcasted_iota(jnp.int32, (BEAMS, NCHUNK), 1))
    b8 = jax.lax.broadcasted_iota(jnp.int32, (BEAMS, 1), 0)
    bw = jax.lax.broadcasted_iota(jnp.int32, (BEAMS, WIN), 0)
    lane = jax.lax.broadcasted_iota(jnp.int32, (1, WIN), 1)

    pool_vals = []
    pool_gidx = []
    for gg in range(GPB):
        r0 = gg * BEAMS
        cmax = cmax_all[r0:r0 + BEAMS, :]
        pgroup = past[r0:r0 + BEAMS, :]
        for _ in range(BEAMS):
            m = jnp.max(cmax)
            sel = jnp.min(jnp.where(cmax == m, pid, BIG_I32))
            bsel = sel // NCHUNK
            csel = sel % NCHUNK
            start = pl.multiple_of(
                jnp.minimum(csel * CHUNK, VOCAB - WIN), 128)
            if r0 < half:
                w = cur_lo_ref[r0:r0 + BEAMS, pl.ds(start, WIN)]
            else:
                w = cur_hi_ref[r0 - half:r0 - half + BEAMS, pl.ds(start, WIN)]
            bias = jnp.max(jnp.where(b8 == bsel, pgroup, NEG_INF))
            row = jnp.max(jnp.where(bw == bsel, w, NEG_INF),
                          axis=0, keepdims=True) + bias
            pool_vals.append(row)
            pool_gidx.append(bsel * VOCAB + start + lane)
            cmax = jnp.where(pid == sel, NEG_INF, cmax)

    vals = jnp.concatenate(pool_vals, axis=0).reshape(GPB, BEAMS, WIN)
    gidx = jnp.concatenate(pool_gidx, axis=0).reshape(GPB, BEAMS, WIN)
    tv = []
    ts = []
    for _ in range(BEAMS):
        m = jnp.max(vals, axis=(1, 2), keepdims=True)
        sel = jnp.min(jnp.where(vals == m, gidx, BIG_I32),
                      axis=(1, 2), keepdims=True)
        tv.append(m)
        ts.append(sel)
        vals = jnp.where(gidx == sel, NEG_INF, vals)
    topv_ref[...] = jnp.concatenate(tv, axis=2)
    sym_ref[...] = jnp.concatenate(ts, axis=2) % VOCAB


@jax.jit
def _run(past_p, cur_p):
    topv, sym = pl.pallas_call(
        _fused,
        grid=(GROUPS // GPB,),
        in_specs=[
            pl.BlockSpec((ROWS, 1), lambda g: (g, 0)),
            pl.BlockSpec((ROWS // 2, VOCAB), lambda g: (2 * g, 0)),
            pl.BlockSpec((ROWS // 2, VOCAB), lambda g: (2 * g + 1, 0)),
        ],
        out_specs=[
            pl.BlockSpec((GPB, 1, BEAMS), lambda g: (g, 0, 0)),
            pl.BlockSpec((GPB, 1, BEAMS), lambda g: (g, 0, 0)),
        ],
        out_shape=(
            jax.ShapeDtypeStruct((GROUPS, 1, BEAMS), jnp.float32),
            jax.ShapeDtypeStruct((GROUPS, 1, BEAMS), jnp.int32),
        ),
        compiler_params=pltpu.CompilerParams(
            dimension_semantics=("parallel",),
        ),
    )(past_p, cur_p, cur_p)

    return topv.reshape(-1, 1), sym.reshape(GROUPS, BEAMS)


def kernel(past_p, cur_p, batch_size, step):
    del batch_size, step
    return _run(past_p, cur_p)


# speedup vs baseline: 1.0305x; 1.0305x over previous
import jax
import jax.numpy as jnp
from jax.experimental import pallas as pl
from jax.experimental.pallas import tpu as pltpu

BEAMS = 8
GROUPS = 32
GPB = 4
ROWS = BEAMS * GPB
VOCAB = 100000
CHUNK = 512
WIN = 544
NCHUNK = (VOCAB + CHUNK - 1) // CHUNK
NEG_INF = float("-inf")
BIG_I32 = 2**31 - 1


def _fused(past_ref, cur_ref, topv_ref, sym_ref):
    maxes = []
    for c in range(NCHUNK):
        lo = c * CHUNK
        hi = min(lo + CHUNK, VOCAB)
        maxes.append(jnp.max(cur_ref[:, lo:hi], axis=1, keepdims=True))
    past = past_ref[...]
    cmax_all = jnp.concatenate(maxes, axis=1) + past

    pid = (jax.lax.broadcasted_iota(jnp.int32, (BEAMS, NCHUNK), 0) * NCHUNK
           + jax.lax.broadcasted_iota(jnp.int32, (BEAMS, NCHUNK), 1))
    b8 = jax.lax.broadcasted_iota(jnp.int32, (BEAMS, 1), 0)
    bw = jax.lax.broadcasted_iota(jnp.int32, (BEAMS, WIN), 0)
    lane = jax.lax.broadcasted_iota(jnp.int32, (1, WIN), 1)

    pool_vals = []
    pool_gidx = []
    for gg in range(GPB):
        r0 = gg * BEAMS
        cmax = cmax_all[r0:r0 + BEAMS, :]
        pgroup = past[r0:r0 + BEAMS, :]
        for _ in range(BEAMS):
            m = jnp.max(cmax)
            sel = jnp.min(jnp.where(cmax == m, pid, BIG_I32))
            bsel = sel // NCHUNK
            csel = sel % NCHUNK
            start = pl.multiple_of(
                jnp.minimum(csel * CHUNK, VOCAB - WIN), 128)
            w = cur_ref[r0:r0 + BEAMS, pl.ds(start, WIN)]
            bias = jnp.max(jnp.where(b8 == bsel, pgroup, NEG_INF))
            row = jnp.max(jnp.where(bw == bsel, w, NEG_INF),
                          axis=0, keepdims=True) + bias
            pool_vals.append(row)
            pool_gidx.append(bsel * VOCAB + start + lane)
            cmax = jnp.where(pid == sel, NEG_INF, cmax)

    vals = jnp.concatenate(pool_vals, axis=0).reshape(GPB, BEAMS, WIN)
    gidx = jnp.concatenate(pool_gidx, axis=0).reshape(GPB, BEAMS, WIN)
    tv = []
    ts = []
    for _ in range(BEAMS):
        m = jnp.max(vals, axis=(1, 2), keepdims=True)
        sel = jnp.min(jnp.where(vals == m, gidx, BIG_I32),
                      axis=(1, 2), keepdims=True)
        tv.append(m)
        ts.append(sel)
        vals = jnp.where(gidx == sel, NEG_INF, vals)
    topv_ref[...] = jnp.concatenate(tv, axis=2)
    sym_ref[...] = jnp.concatenate(ts, axis=2) % VOCAB


@jax.jit
def _run(past_p, cur_p):
    topv, sym = pl.pallas_call(
        _fused,
        grid=(GROUPS // GPB,),
        in_specs=[
            pl.BlockSpec((ROWS, 1), lambda g: (g, 0)),
            pl.BlockSpec((ROWS, VOCAB), lambda g: (g, 0)),
        ],
        out_specs=[
            pl.BlockSpec((GPB, 1, BEAMS), lambda g: (g, 0, 0)),
            pl.BlockSpec((GPB, 1, BEAMS), lambda g: (g, 0, 0)),
        ],
        out_shape=(
            jax.ShapeDtypeStruct((GROUPS, 1, BEAMS), jnp.float32),
            jax.ShapeDtypeStruct((GROUPS, 1, BEAMS), jnp.int32),
        ),
        compiler_params=pltpu.CompilerParams(
            dimension_semantics=("parallel",),
        ),
    )(past_p, cur_p)

    return topv.reshape(-1, 1), sym.reshape(GROUPS, BEAMS)


def kernel(past_p, cur_p, batch_size, step):
    del batch_size, step
    return _run(past_p, cur_p)


# speedup vs baseline: 1.0631x; 1.0316x over previous
import jax
import jax.numpy as jnp
from jax.experimental import pallas as pl
from jax.experimental.pallas import tpu as pltpu

BEAMS = 8
GROUPS = 32
GPB = 8
ROWS = BEAMS * GPB
VOCAB = 100000
CHUNK = 512
WIN = 544
NCHUNK = (VOCAB + CHUNK - 1) // CHUNK
NEG_INF = float("-inf")
BIG_I32 = 2**31 - 1


def _fused(past_ref, cur_ref, topv_ref, sym_ref):
    maxes = []
    for c in range(NCHUNK):
        lo = c * CHUNK
        hi = min(lo + CHUNK, VOCAB)
        maxes.append(jnp.max(cur_ref[:, lo:hi], axis=1, keepdims=True))
    past = past_ref[...]
    cmax_all = jnp.concatenate(maxes, axis=1) + past

    pid = (jax.lax.broadcasted_iota(jnp.int32, (BEAMS, NCHUNK), 0) * NCHUNK
           + jax.lax.broadcasted_iota(jnp.int32, (BEAMS, NCHUNK), 1))
    b8 = jax.lax.broadcasted_iota(jnp.int32, (BEAMS, 1), 0)
    bw = jax.lax.broadcasted_iota(jnp.int32, (BEAMS, WIN), 0)
    lane = jax.lax.broadcasted_iota(jnp.int32, (1, WIN), 1)

    pool_vals = []
    pool_gidx = []
    for gg in range(GPB):
        r0 = gg * BEAMS
        cmax = cmax_all[r0:r0 + BEAMS, :]
        pgroup = past[r0:r0 + BEAMS, :]
        for _ in range(BEAMS):
            m = jnp.max(cmax)
            sel = jnp.min(jnp.where(cmax == m, pid, BIG_I32))
            bsel = sel // NCHUNK
            csel = sel % NCHUNK
            start = pl.multiple_of(
                jnp.minimum(csel * CHUNK, VOCAB - WIN), 128)
            w = cur_ref[r0:r0 + BEAMS, pl.ds(start, WIN)]
            bias = jnp.max(jnp.where(b8 == bsel, pgroup, NEG_INF))
            row = jnp.max(jnp.where(bw == bsel, w, NEG_INF),
                          axis=0, keepdims=True) + bias
            pool_vals.append(row)
            pool_gidx.append(bsel * VOCAB + start + lane)
            cmax = jnp.where(pid == sel, NEG_INF, cmax)

    vals = jnp.concatenate(pool_vals, axis=0).reshape(GPB, BEAMS, WIN)
    gidx = jnp.concatenate(pool_gidx, axis=0).reshape(GPB, BEAMS, WIN)
    tv = []
    ts = []
    for _ in range(BEAMS):
        m = jnp.max(vals, axis=(1, 2), keepdims=True)
        sel = jnp.min(jnp.where(vals == m, gidx, BIG_I32),
                      axis=(1, 2), keepdims=True)
        tv.append(m)
        ts.append(sel)
        vals = jnp.where(gidx == sel, NEG_INF, vals)
    topv_ref[...] = jnp.concatenate(tv, axis=2)
    sym_ref[...] = jnp.concatenate(ts, axis=2) % VOCAB


@jax.jit
def _run(past_p, cur_p):
    topv, sym = pl.pallas_call(
        _fused,
        grid=(GROUPS // GPB,),
        in_specs=[
            pl.BlockSpec((ROWS, 1), lambda g: (g, 0)),
            pl.BlockSpec((ROWS, VOCAB), lambda g: (g, 0)),
        ],
        out_specs=[
            pl.BlockSpec((GPB, 1, BEAMS), lambda g: (g, 0, 0)),
            pl.BlockSpec((GPB, 1, BEAMS), lambda g: (g, 0, 0)),
        ],
        out_shape=(
            jax.ShapeDtypeStruct((GROUPS, 1, BEAMS), jnp.float32),
            jax.ShapeDtypeStruct((GROUPS, 1, BEAMS), jnp.int32),
        ),
        compiler_params=pltpu.CompilerParams(
            dimension_semantics=("parallel",),
        ),
    )(past_p, cur_p)

    return topv.reshape(-1, 1), sym.reshape(GROUPS, BEAMS)


def kernel(past_p, cur_p, batch_size, step):
    del batch_size, step
    return _run(past_p, cur_p)
